# initial kernel scaffold (unmeasured)
import functools

import jax
import jax.numpy as jnp
from jax import lax
from jax.experimental import pallas as pl
from jax.experimental.pallas import tpu as pltpu

N_DEV = 16


def kernel(x, router_W, route_idx, expert_W, shared_W):
    T, D = x.shape
    E_LOC, _, H = expert_W.shape
    E_TOT = router_W.shape[1]
    n_hops = N_DEV - 1

    def body(x_ref, rW_ref, idx_ref, eW_ref, sW_ref, out_ref,
             comm_ref, send_sems, recv_sems):
        my = lax.axis_index("i")
        left = lax.rem(my + (N_DEV - 1), N_DEV)
        right = lax.rem(my + 1, N_DEV)

        barrier_sem = pltpu.get_barrier_semaphore()
        for nbr in (left, right):
            pl.semaphore_signal(barrier_sem, inc=1, device_id=(nbr,),
                                device_id_type=pl.DeviceIdType.MESH)
        pl.semaphore_wait(barrier_sem, 2)

        xv = x_ref[:, :]
        scores = jnp.dot(xv, rW_ref[:, :], preferred_element_type=jnp.float32)
        m = jnp.max(scores, axis=-1, keepdims=True)
        ex = jnp.exp(scores - m)
        probs = ex / jnp.sum(ex, axis=-1, keepdims=True)
        eidx = idx_ref[:, :]
        one_hot = (jax.lax.broadcasted_iota(jnp.int32, (T, E_TOT), 1)
                   == eidx)
        gp = jnp.sum(jnp.where(one_hot, probs, 0.0), axis=-1,
                     keepdims=True)

        acc = jnp.dot(xv, sW_ref[:, :], preferred_element_type=jnp.float32)

        for h in range(N_DEV):
            if h < n_hops:
                rdma = pltpu.make_async_remote_copy(
                    src_ref=(eW_ref if h == 0 else comm_ref.at[h - 1]),
                    dst_ref=comm_ref.at[h],
                    send_sem=send_sems.at[h],
                    recv_sem=recv_sems.at[h],
                    device_id=(right,),
                    device_id_type=pl.DeviceIdType.MESH,
                )
                rdma.start()

            src_dev = lax.rem(my + (N_DEV - h), N_DEV)
            for j in range(E_LOC):
                gid = src_dev * E_LOC + j
                coeff = jnp.where(eidx == gid, gp, 0.0)
                xg = xv * coeff
                wj = eW_ref[j] if h == 0 else comm_ref[h - 1, j]
                acc = acc + jnp.dot(xg, wj,
                                    preferred_element_type=jnp.float32)

            if h < n_hops:
                rdma.wait()

        out_ref[:, :] = acc

        @functools.partial(pl.run_scoped,
                           second_barrier=pltpu.SemaphoreType.REGULAR)
        def _(second_barrier):
            for nbr in (left, right):
                pl.semaphore_signal(second_barrier, inc=1, device_id=(nbr,),
                                    device_id_type=pl.DeviceIdType.MESH)
            pl.semaphore_wait(second_barrier, 2)

    return pl.pallas_call(
        body,
        out_shape=jax.ShapeDtypeStruct((T, H), jnp.float32),
        in_specs=[pl.BlockSpec(memory_space=pltpu.VMEM)] * 5,
        out_specs=pl.BlockSpec(memory_space=pltpu.VMEM),
        scratch_shapes=[
            pltpu.VMEM((n_hops, E_LOC, D, H), jnp.float32),
            pltpu.SemaphoreType.DMA((n_hops,)),
            pltpu.SemaphoreType.DMA((n_hops,)),
        ],
        compiler_params=pltpu.CompilerParams(collective_id=0),
    )(x, router_W, route_idx, expert_W, shared_W)


# baseline (device time: 380117 ns/iter reference)
import functools

import jax
import jax.numpy as jnp
from jax import lax
from jax.experimental import pallas as pl
from jax.experimental.pallas import tpu as pltpu

N_DEV = 16


def kernel(x, router_W, route_idx, expert_W, shared_W):
    T, D = x.shape
    E_LOC, _, H = expert_W.shape
    E_TOT = router_W.shape[1]
    n_hops = N_DEV - 1

    def body(x_ref, rW_ref, idx_ref, eW_ref, sW_ref, out_ref,
             comm_ref, send_sems, recv_sems):
        my = lax.axis_index("i")
        left = lax.rem(my + (N_DEV - 1), N_DEV)
        right = lax.rem(my + 1, N_DEV)

        barrier_sem = pltpu.get_barrier_semaphore()
        for nbr in (left, right):
            pl.semaphore_signal(barrier_sem, inc=1, device_id=(nbr,),
                                device_id_type=pl.DeviceIdType.MESH)
        pl.semaphore_wait(barrier_sem, 2)

        xv = x_ref[:, :]
        scores = jnp.dot(xv, rW_ref[:, :], preferred_element_type=jnp.float32)
        m = jnp.max(scores, axis=-1, keepdims=True)
        ex = jnp.exp(scores - m)
        probs = ex / jnp.sum(ex, axis=-1, keepdims=True)
        eidx = idx_ref[:, :]
        one_hot = (jax.lax.broadcasted_iota(jnp.int32, (T, E_TOT), 1)
                   == eidx)
        gp = jnp.sum(jnp.where(one_hot, probs, 0.0), axis=-1,
                     keepdims=True)

        acc = jnp.dot(xv, sW_ref[:, :], preferred_element_type=jnp.float32)

        for h in range(N_DEV):
            if h < n_hops:
                rdma = pltpu.make_async_remote_copy(
                    src_ref=(eW_ref if h == 0 else comm_ref.at[h - 1]),
                    dst_ref=comm_ref.at[h],
                    send_sem=send_sems.at[h],
                    recv_sem=recv_sems.at[h],
                    device_id=(right,),
                    device_id_type=pl.DeviceIdType.MESH,
                )
                rdma.start()

            src_dev = lax.rem(my + (N_DEV - h), N_DEV)
            for j in range(E_LOC):
                gid = src_dev * E_LOC + j
                coeff = jnp.where(eidx == gid, gp, 0.0)
                xg = xv * coeff
                wj = eW_ref[j] if h == 0 else comm_ref[h - 1, j]
                acc = acc + jnp.dot(xg, wj,
                                    preferred_element_type=jnp.float32)

            if h < n_hops:
                rdma.wait()

        out_ref[:, :] = acc

        @functools.partial(pl.run_scoped,
                           second_barrier=pltpu.SemaphoreType.REGULAR)
        def _(second_barrier):
            for nbr in (left, right):
                pl.semaphore_signal(second_barrier, inc=1, device_id=(nbr,),
                                    device_id_type=pl.DeviceIdType.MESH)
            pl.semaphore_wait(second_barrier, 2)

    return pl.pallas_call(
        body,
        out_shape=jax.ShapeDtypeStruct((T, H), jnp.float32),
        in_specs=[pl.BlockSpec(memory_space=pltpu.VMEM)] * 5,
        out_specs=pl.BlockSpec(memory_space=pltpu.VMEM),
        scratch_shapes=[
            pltpu.VMEM((n_hops, E_LOC, D, H), jnp.float32),
            pltpu.SemaphoreType.DMA((n_hops,)),
            pltpu.SemaphoreType.DMA((n_hops,)),
        ],
        compiler_params=pltpu.CompilerParams(
            collective_id=0,
            vmem_limit_bytes=96 * 1024 * 1024,
        ),
    )(x, router_W, route_idx, expert_W, shared_W)


# device time: 101073 ns/iter; 3.7608x vs baseline; 3.7608x over previous
import jax
import jax.numpy as jnp
from jax import lax
from jax.experimental import pallas as pl
from jax.experimental.pallas import tpu as pltpu

N_DEV = 16
CAP = 128


def kernel(x, router_W, route_idx, expert_W, shared_W):
    T, D = x.shape
    E_LOC, _, H = expert_W.shape
    E_TOT = router_W.shape[1]
    f32 = jnp.float32

    def body(x_ref, rW_ref, idx_ref, eW_ref, sW_ref, out_ref,
             pack_ref, disp_ref, yout_ref, yin_ref,
             dsend, drecv, ysend, yrecv):
        my = lax.axis_index("i")

        barrier_sem = pltpu.get_barrier_semaphore()
        for k in range(1, N_DEV):
            peer = lax.rem(my + k, N_DEV)
            pl.semaphore_signal(barrier_sem, inc=1, device_id=(peer,),
                                device_id_type=pl.DeviceIdType.MESH)
        pl.semaphore_wait(barrier_sem, N_DEV - 1)

        xv = x_ref[:, :]
        scores = jnp.dot(xv, rW_ref[:, :], preferred_element_type=f32,
                         precision=lax.Precision.HIGHEST)
        mx = jnp.max(scores, axis=-1, keepdims=True)
        ex = jnp.exp(scores - mx)
        probs = ex / jnp.sum(ex, axis=-1, keepdims=True)
        eidx = idx_ref[:, :]
        oh_tok = lax.broadcasted_iota(jnp.int32, (T, E_TOT), 1) == eidx
        gp = jnp.sum(jnp.where(oh_tok, probs, 0.0), axis=-1,
                     keepdims=True)
        xg = xv * gp

        dst_tok = eidx // E_LOC

        iota16 = lax.broadcasted_iota(jnp.int32, (T, N_DEV), 1)
        Mf = jnp.where(iota16 == dst_tok, 1.0, 0.0)
        ir = lax.broadcasted_iota(jnp.int32, (T, T), 0)
        ic = lax.broadcasted_iota(jnp.int32, (T, T), 1)
        tri = jnp.where(ic <= ir, 1.0, 0.0)
        prefixs = jnp.dot(tri, Mf, preferred_element_type=f32)

        iota_cap = lax.broadcasted_iota(jnp.int32, (T, CAP), 1)

        def make_Pt(d):
            mask_d = jnp.where(dst_tok == d, 1.0, 0.0)
            pre_d = jnp.sum(jnp.where(iota16 == d, prefixs, 0.0),
                            axis=-1, keepdims=True)
            pre_i = pre_d.astype(jnp.int32)
            return jnp.where(iota_cap == pre_i - 1, mask_d, 0.0)

        disp_rdmas = []
        for k in range(1, N_DEV):
            d = lax.rem(my + k, N_DEV)
            Pt = make_Pt(d)
            xp = lax.dot_general(Pt, xg, (((0,), (0,)), ((), ())),
                                 preferred_element_type=f32)
            pack_ref[k - 1] = xp
            rdma = pltpu.make_async_remote_copy(
                src_ref=pack_ref.at[k - 1],
                dst_ref=disp_ref.at[k - 1],
                send_sem=dsend.at[k - 1],
                recv_sem=drecv.at[k - 1],
                device_id=(d,),
                device_id_type=pl.DeviceIdType.MESH,
            )
            rdma.start()
            disp_rdmas.append(rdma)

        acc = jnp.dot(xv, sW_ref[:, :], preferred_element_type=f32)
        for j in range(E_LOC):
            gid = my * E_LOC + j
            cj = jnp.where(eidx == gid, gp, 0.0)
            acc = acc + jnp.dot(xv * cj, eW_ref[j],
                                preferred_element_type=f32)

        iota64c = lax.broadcasted_iota(jnp.int32, (CAP, E_TOT), 1)
        y_rdmas = []
        for k in range(1, N_DEV):
            src = lax.rem(my + (N_DEV - k), N_DEV)
            recv = pltpu.make_async_remote_copy(
                src_ref=pack_ref.at[k - 1],
                dst_ref=disp_ref.at[k - 1],
                send_sem=dsend.at[k - 1],
                recv_sem=drecv.at[k - 1],
                device_id=(src,),
                device_id_type=pl.DeviceIdType.MESH,
            )
            recv.wait_recv()
            xr = disp_ref[k - 1]
            s_full = jnp.dot(xr, rW_ref[:, :], preferred_element_type=f32,
                             precision=lax.Precision.HIGHEST)
            valid = (iota64c // E_LOC) == my
            s_m = jnp.where(valid, s_full, -jnp.inf)
            rmax = jnp.max(s_m, axis=-1, keepdims=True)
            oh = jnp.where(valid & (s_m == rmax), 1.0, 0.0)
            y = jnp.zeros((CAP, H), f32)
            for j in range(E_LOC):
                gid = my * E_LOC + j
                cj = jnp.sum(jnp.where(iota64c == gid, oh, 0.0),
                             axis=-1, keepdims=True)
                y = y + jnp.dot(xr * cj, eW_ref[j],
                                preferred_element_type=f32)
            yout_ref[k - 1] = y
            rdma = pltpu.make_async_remote_copy(
                src_ref=yout_ref.at[k - 1],
                dst_ref=yin_ref.at[k - 1],
                send_sem=ysend.at[k - 1],
                recv_sem=yrecv.at[k - 1],
                device_id=(src,),
                device_id_type=pl.DeviceIdType.MESH,
            )
            rdma.start()
            y_rdmas.append(rdma)

        for k in range(1, N_DEV):
            d = lax.rem(my + k, N_DEV)
            recv = pltpu.make_async_remote_copy(
                src_ref=yout_ref.at[k - 1],
                dst_ref=yin_ref.at[k - 1],
                send_sem=ysend.at[k - 1],
                recv_sem=yrecv.at[k - 1],
                device_id=(d,),
                device_id_type=pl.DeviceIdType.MESH,
            )
            recv.wait_recv()
            Pt = make_Pt(d)
            acc = acc + jnp.dot(Pt, yin_ref[k - 1],
                                preferred_element_type=f32)

        out_ref[:, :] = acc

        for r in disp_rdmas:
            r.wait_send()
        for r in y_rdmas:
            r.wait_send()

    return pl.pallas_call(
        body,
        out_shape=jax.ShapeDtypeStruct((T, H), jnp.float32),
        in_specs=[pl.BlockSpec(memory_space=pltpu.VMEM)] * 5,
        out_specs=pl.BlockSpec(memory_space=pltpu.VMEM),
        scratch_shapes=[
            pltpu.VMEM((N_DEV - 1, CAP, D), jnp.float32),
            pltpu.VMEM((N_DEV - 1, CAP, D), jnp.float32),
            pltpu.VMEM((N_DEV - 1, CAP, H), jnp.float32),
            pltpu.VMEM((N_DEV - 1, CAP, H), jnp.float32),
            pltpu.SemaphoreType.DMA((N_DEV - 1,)),
            pltpu.SemaphoreType.DMA((N_DEV - 1,)),
            pltpu.SemaphoreType.DMA((N_DEV - 1,)),
            pltpu.SemaphoreType.DMA((N_DEV - 1,)),
        ],
        compiler_params=pltpu.CompilerParams(
            collective_id=0,
            vmem_limit_bytes=96 * 1024 * 1024,
        ),
    )(x, router_W, route_idx, expert_W, shared_W)


# device time: 98281 ns/iter; 3.8677x vs baseline; 1.0284x over previous
import jax
import jax.numpy as jnp
from jax import lax
from jax.experimental import pallas as pl
from jax.experimental.pallas import tpu as pltpu

N_DEV = 16
CAP = 128


def kernel(x, router_W, route_idx, expert_W, shared_W):
    T, D = x.shape
    E_LOC, _, H = expert_W.shape
    E_TOT = router_W.shape[1]
    f32 = jnp.float32

    def body(x_ref, rW_ref, idx_ref, eW_ref, sW_ref, out_ref,
             pack_ref, disp_ref, yout_ref, yin_ref,
             dsend, drecv, ysend, yrecv):
        my = lax.axis_index("i")

        barrier_sem = pltpu.get_barrier_semaphore()
        for k in range(1, N_DEV):
            peer = lax.rem(my + k, N_DEV)
            pl.semaphore_signal(barrier_sem, inc=1, device_id=(peer,),
                                device_id_type=pl.DeviceIdType.MESH)
        pl.semaphore_wait(barrier_sem, N_DEV - 1)

        xv = x_ref[:, :]
        scores = jnp.dot(xv, rW_ref[:, :], preferred_element_type=f32,
                         precision=lax.Precision.HIGHEST)
        mx = jnp.max(scores, axis=-1, keepdims=True)
        ex = jnp.exp(scores - mx)
        probs = ex / jnp.sum(ex, axis=-1, keepdims=True)
        eidx = idx_ref[:, :]
        oh_tok = lax.broadcasted_iota(jnp.int32, (T, E_TOT), 1) == eidx
        gp = jnp.sum(jnp.where(oh_tok, probs, 0.0), axis=-1,
                     keepdims=True)
        xg = xv * gp

        dst_tok = eidx // E_LOC

        iota16 = lax.broadcasted_iota(jnp.int32, (T, N_DEV), 1)
        Mf = jnp.where(iota16 == dst_tok, 1.0, 0.0)
        ir = lax.broadcasted_iota(jnp.int32, (T, T), 0)
        ic = lax.broadcasted_iota(jnp.int32, (T, T), 1)
        tri = jnp.where(ic <= ir, 1.0, 0.0)
        prefixs = jnp.dot(tri, Mf, preferred_element_type=f32)

        iota_cap = lax.broadcasted_iota(jnp.int32, (T, CAP), 1)

        def make_Pt(d):
            mask_d = jnp.where(dst_tok == d, 1.0, 0.0)
            pre_d = jnp.sum(jnp.where(iota16 == d, prefixs, 0.0),
                            axis=-1, keepdims=True)
            pre_i = pre_d.astype(jnp.int32)
            return jnp.where(iota_cap == pre_i - 1, mask_d, 0.0)

        near_first = sorted(range(1, N_DEV), key=lambda k: min(k, N_DEV - k))
        far_first = near_first[::-1]

        disp_rdmas = []
        for k in far_first:
            d = lax.rem(my + k, N_DEV)
            Pt = make_Pt(d)
            xp = lax.dot_general(Pt, xg, (((0,), (0,)), ((), ())),
                                 preferred_element_type=f32)
            pack_ref[k - 1] = xp
            rdma = pltpu.make_async_remote_copy(
                src_ref=pack_ref.at[k - 1],
                dst_ref=disp_ref.at[k - 1],
                send_sem=dsend.at[k - 1],
                recv_sem=drecv.at[k - 1],
                device_id=(d,),
                device_id_type=pl.DeviceIdType.MESH,
            )
            rdma.start()
            disp_rdmas.append(rdma)

        acc = jnp.dot(xv, sW_ref[:, :], preferred_element_type=f32)
        for j in range(E_LOC):
            gid = my * E_LOC + j
            cj = jnp.where(eidx == gid, gp, 0.0)
            acc = acc + jnp.dot(xv * cj, eW_ref[j],
                                preferred_element_type=f32)

        iota64c = lax.broadcasted_iota(jnp.int32, (CAP, E_TOT), 1)
        y_rdmas = []
        for k in near_first:
            src = lax.rem(my + (N_DEV - k), N_DEV)
            recv = pltpu.make_async_remote_copy(
                src_ref=pack_ref.at[k - 1],
                dst_ref=disp_ref.at[k - 1],
                send_sem=dsend.at[k - 1],
                recv_sem=drecv.at[k - 1],
                device_id=(src,),
                device_id_type=pl.DeviceIdType.MESH,
            )
            recv.wait_recv()
            xr = disp_ref[k - 1]
            s_full = jnp.dot(xr, rW_ref[:, :], preferred_element_type=f32,
                             precision=lax.Precision.HIGHEST)
            valid = (iota64c // E_LOC) == my
            s_m = jnp.where(valid, s_full, -jnp.inf)
            rmax = jnp.max(s_m, axis=-1, keepdims=True)
            oh = jnp.where(valid & (s_m == rmax), 1.0, 0.0)
            y = jnp.zeros((CAP, H), f32)
            for j in range(E_LOC):
                gid = my * E_LOC + j
                cj = jnp.sum(jnp.where(iota64c == gid, oh, 0.0),
                             axis=-1, keepdims=True)
                y = y + jnp.dot(xr * cj, eW_ref[j],
                                preferred_element_type=f32)
            yout_ref[k - 1] = y
            rdma = pltpu.make_async_remote_copy(
                src_ref=yout_ref.at[k - 1],
                dst_ref=yin_ref.at[k - 1],
                send_sem=ysend.at[k - 1],
                recv_sem=yrecv.at[k - 1],
                device_id=(src,),
                device_id_type=pl.DeviceIdType.MESH,
            )
            rdma.start()
            y_rdmas.append(rdma)

        for k in near_first:
            d = lax.rem(my + k, N_DEV)
            recv = pltpu.make_async_remote_copy(
                src_ref=yout_ref.at[k - 1],
                dst_ref=yin_ref.at[k - 1],
                send_sem=ysend.at[k - 1],
                recv_sem=yrecv.at[k - 1],
                device_id=(d,),
                device_id_type=pl.DeviceIdType.MESH,
            )
            recv.wait_recv()
            Pt = make_Pt(d)
            acc = acc + jnp.dot(Pt, yin_ref[k - 1],
                                preferred_element_type=f32)

        out_ref[:, :] = acc

        for r in disp_rdmas:
            r.wait_send()
        for r in y_rdmas:
            r.wait_send()

    return pl.pallas_call(
        body,
        out_shape=jax.ShapeDtypeStruct((T, H), jnp.float32),
        in_specs=[pl.BlockSpec(memory_space=pltpu.VMEM)] * 5,
        out_specs=pl.BlockSpec(memory_space=pltpu.VMEM),
        scratch_shapes=[
            pltpu.VMEM((N_DEV - 1, CAP, D), jnp.float32),
            pltpu.VMEM((N_DEV - 1, CAP, D), jnp.float32),
            pltpu.VMEM((N_DEV - 1, CAP, H), jnp.float32),
            pltpu.VMEM((N_DEV - 1, CAP, H), jnp.float32),
            pltpu.SemaphoreType.DMA((N_DEV - 1,)),
            pltpu.SemaphoreType.DMA((N_DEV - 1,)),
            pltpu.SemaphoreType.DMA((N_DEV - 1,)),
            pltpu.SemaphoreType.DMA((N_DEV - 1,)),
        ],
        compiler_params=pltpu.CompilerParams(
            collective_id=0,
            vmem_limit_bytes=96 * 1024 * 1024,
        ),
    )(x, router_W, route_idx, expert_W, shared_W)


# device time: 79123 ns/iter; 4.8041x vs baseline; 1.2421x over previous
import jax
import jax.numpy as jnp
from jax import lax
from jax.experimental import pallas as pl
from jax.experimental.pallas import tpu as pltpu

N_DEV = 16
CAP = 128


def kernel(x, router_W, route_idx, expert_W, shared_W):
    T, D = x.shape
    E_LOC, _, H = expert_W.shape
    E_TOT = router_W.shape[1]
    f32 = jnp.float32
    bf16 = jnp.bfloat16

    def body(x_ref, rW_ref, idx_ref, eW_ref, sW_ref, out_ref,
             pack_ref, disp_ref, yout_ref, yin_ref,
             dsend, drecv, ysend, yrecv):
        my = lax.axis_index("i")

        barrier_sem = pltpu.get_barrier_semaphore()
        for k in range(1, N_DEV):
            peer = lax.rem(my + k, N_DEV)
            pl.semaphore_signal(barrier_sem, inc=1, device_id=(peer,),
                                device_id_type=pl.DeviceIdType.MESH)
        pl.semaphore_wait(barrier_sem, N_DEV - 1)

        xv = x_ref[:, :]
        scores = jnp.dot(xv, rW_ref[:, :], preferred_element_type=f32,
                         precision=lax.Precision.HIGHEST)
        mx = jnp.max(scores, axis=-1, keepdims=True)
        ex = jnp.exp(scores - mx)
        probs = ex / jnp.sum(ex, axis=-1, keepdims=True)
        eidx = idx_ref[:, :]
        oh_tok = lax.broadcasted_iota(jnp.int32, (T, E_TOT), 1) == eidx
        gp = jnp.sum(jnp.where(oh_tok, probs, 0.0), axis=-1,
                     keepdims=True)
        xg = xv * gp

        dst_tok = eidx // E_LOC

        iota16 = lax.broadcasted_iota(jnp.int32, (T, N_DEV), 1)
        Mf = jnp.where(iota16 == dst_tok, 1.0, 0.0)
        ir = lax.broadcasted_iota(jnp.int32, (T, T), 0)
        ic = lax.broadcasted_iota(jnp.int32, (T, T), 1)
        tri = jnp.where(ic <= ir, 1.0, 0.0)
        prefixs = jnp.dot(tri, Mf, preferred_element_type=f32)

        iota_cap = lax.broadcasted_iota(jnp.int32, (T, CAP), 1)

        def make_Pt(d):
            mask_d = jnp.where(dst_tok == d, 1.0, 0.0)
            pre_d = jnp.sum(jnp.where(iota16 == d, prefixs, 0.0),
                            axis=-1, keepdims=True)
            pre_i = pre_d.astype(jnp.int32)
            return jnp.where(iota_cap == pre_i - 1, mask_d, 0.0)

        near_first = sorted(range(1, N_DEV), key=lambda k: min(k, N_DEV - k))
        far_first = near_first[::-1]

        PtAll = jnp.concatenate(
            [make_Pt(lax.rem(my + k, N_DEV)) for k in range(1, N_DEV)],
            axis=1)
        xpAll = lax.dot_general(PtAll, xg, (((0,), (0,)), ((), ())),
                                preferred_element_type=f32)
        disp_rdmas = []
        for k in far_first:
            pack_ref[k - 1] = xpAll[(k - 1) * CAP:k * CAP]
            rdma = pltpu.make_async_remote_copy(
                src_ref=pack_ref.at[k - 1],
                dst_ref=disp_ref.at[k - 1],
                send_sem=dsend.at[k - 1],
                recv_sem=drecv.at[k - 1],
                device_id=(lax.rem(my + k, N_DEV),),
                device_id_type=pl.DeviceIdType.MESH,
            )
            rdma.start()
            disp_rdmas.append(rdma)

        acc = jnp.dot(xv, sW_ref[:, :], preferred_element_type=f32)
        for j in range(E_LOC):
            gid = my * E_LOC + j
            cj = jnp.where(eidx == gid, gp, 0.0)
            acc = acc + jnp.dot(xv * cj, eW_ref[j],
                                preferred_element_type=f32)

        iota64c = lax.broadcasted_iota(jnp.int32, (CAP, E_TOT), 1)
        y_rdmas = []
        for k in near_first:
            src = lax.rem(my + (N_DEV - k), N_DEV)
            recv = pltpu.make_async_remote_copy(
                src_ref=pack_ref.at[k - 1],
                dst_ref=disp_ref.at[k - 1],
                send_sem=dsend.at[k - 1],
                recv_sem=drecv.at[k - 1],
                device_id=(src,),
                device_id_type=pl.DeviceIdType.MESH,
            )
            recv.wait_recv()
            xr = disp_ref[k - 1]
            s_full = jnp.dot(xr, rW_ref[:, :], preferred_element_type=f32,
                             precision=lax.Precision.HIGHEST)
            valid = (iota64c // E_LOC) == my
            s_m = jnp.where(valid, s_full, -jnp.inf)
            rmax = jnp.max(s_m, axis=-1, keepdims=True)
            oh = jnp.where(valid & (s_m == rmax), 1.0, 0.0)
            y = jnp.zeros((CAP, H), f32)
            for j in range(E_LOC):
                gid = my * E_LOC + j
                cj = jnp.sum(jnp.where(iota64c == gid, oh, 0.0),
                             axis=-1, keepdims=True)
                y = y + jnp.dot(xr * cj, eW_ref[j],
                                preferred_element_type=f32)
            yout_ref[k - 1] = y.astype(bf16)
            rdma = pltpu.make_async_remote_copy(
                src_ref=yout_ref.at[k - 1],
                dst_ref=yin_ref.at[k - 1],
                send_sem=ysend.at[k - 1],
                recv_sem=yrecv.at[k - 1],
                device_id=(src,),
                device_id_type=pl.DeviceIdType.MESH,
            )
            rdma.start()
            y_rdmas.append(rdma)

        for k in near_first:
            recv = pltpu.make_async_remote_copy(
                src_ref=yout_ref.at[k - 1],
                dst_ref=yin_ref.at[k - 1],
                send_sem=ysend.at[k - 1],
                recv_sem=yrecv.at[k - 1],
                device_id=(lax.rem(my + k, N_DEV),),
                device_id_type=pl.DeviceIdType.MESH,
            )
            recv.wait_recv()
        ysAll = jnp.concatenate(
            [yin_ref[k - 1].astype(f32) for k in range(1, N_DEV)],
            axis=0)
        acc = acc + jnp.dot(PtAll, ysAll, preferred_element_type=f32)

        out_ref[:, :] = acc

        for r in disp_rdmas:
            r.wait_send()
        for r in y_rdmas:
            r.wait_send()

    return pl.pallas_call(
        body,
        out_shape=jax.ShapeDtypeStruct((T, H), jnp.float32),
        in_specs=[pl.BlockSpec(memory_space=pltpu.VMEM)] * 5,
        out_specs=pl.BlockSpec(memory_space=pltpu.VMEM),
        scratch_shapes=[
            pltpu.VMEM((N_DEV - 1, CAP, D), jnp.float32),
            pltpu.VMEM((N_DEV - 1, CAP, D), jnp.float32),
            pltpu.VMEM((N_DEV - 1, CAP, H), jnp.bfloat16),
            pltpu.VMEM((N_DEV - 1, CAP, H), jnp.bfloat16),
            pltpu.SemaphoreType.DMA((N_DEV - 1,)),
            pltpu.SemaphoreType.DMA((N_DEV - 1,)),
            pltpu.SemaphoreType.DMA((N_DEV - 1,)),
            pltpu.SemaphoreType.DMA((N_DEV - 1,)),
        ],
        compiler_params=pltpu.CompilerParams(
            collective_id=0,
            vmem_limit_bytes=96 * 1024 * 1024,
        ),
    )(x, router_W, route_idx, expert_W, shared_W)


# device time: 74025 ns/iter; 5.1350x vs baseline; 1.0689x over previous
import jax
import jax.numpy as jnp
from jax import lax
from jax.experimental import pallas as pl
from jax.experimental.pallas import tpu as pltpu

N_DEV = 16
CAP = 128
PAD_L = 384


def kernel(x, router_W, route_idx, expert_W, shared_W):
    T, D = x.shape
    E_LOC, _, H = expert_W.shape
    E_TOT = router_W.shape[1]
    f32 = jnp.float32
    bf16 = jnp.bfloat16

    def body(x_ref, rW_ref, idx_ref, eW_ref, sW_ref, out_ref,
             pack_ref, disp_ref, yout_ref, yin_ref,
             dsend, drecv, ysend, yrecv):
        my = lax.axis_index("i")

        barrier_sem = pltpu.get_barrier_semaphore()
        for k in range(1, N_DEV):
            peer = lax.rem(my + k, N_DEV)
            pl.semaphore_signal(barrier_sem, inc=1, device_id=(peer,),
                                device_id_type=pl.DeviceIdType.MESH)
        pl.semaphore_wait(barrier_sem, N_DEV - 1)

        xv = x_ref[:, :]
        scores = jnp.dot(xv, rW_ref[:, :], preferred_element_type=f32,
                         precision=lax.Precision.HIGHEST)
        mx = jnp.max(scores, axis=-1, keepdims=True)
        ex = jnp.exp(scores - mx)
        probs = ex / jnp.sum(ex, axis=-1, keepdims=True)
        eidx = idx_ref[:, :]
        oh_tok = lax.broadcasted_iota(jnp.int32, (T, E_TOT), 1) == eidx
        gp = jnp.sum(jnp.where(oh_tok, probs, 0.0), axis=-1,
                     keepdims=True)
        xg = xv * gp

        dst_tok = eidx // E_LOC

        iota16 = lax.broadcasted_iota(jnp.int32, (T, N_DEV), 1)
        Mf = jnp.where(iota16 == dst_tok, 1.0, 0.0)
        ir = lax.broadcasted_iota(jnp.int32, (T, T), 0)
        ic = lax.broadcasted_iota(jnp.int32, (T, T), 1)
        tri = jnp.where(ic <= ir, 1.0, 0.0)
        prefixs = jnp.dot(tri, Mf, preferred_element_type=f32)

        iota_cap = lax.broadcasted_iota(jnp.int32, (T, CAP), 1)

        def make_Pt(d):
            mask_d = jnp.where(dst_tok == d, 1.0, 0.0)
            pre_d = jnp.sum(jnp.where(iota16 == d, prefixs, 0.0),
                            axis=-1, keepdims=True)
            pre_i = pre_d.astype(jnp.int32)
            return jnp.where(iota_cap == pre_i - 1, mask_d, 0.0)

        near_first = sorted(range(1, N_DEV), key=lambda k: min(k, N_DEV - k))
        far_first = near_first[::-1]

        PtAll = jnp.concatenate(
            [make_Pt(lax.rem(my + k, N_DEV)) for k in range(1, N_DEV)],
            axis=1)
        xpAll = lax.dot_general(PtAll, xg, (((0,), (0,)), ((), ())),
                                preferred_element_type=f32)
        jloc_oh = jnp.where(
            lax.broadcasted_iota(jnp.int32, (T, E_LOC), 1) == eidx % E_LOC,
            1.0, 0.0)
        JAll = lax.dot_general(PtAll, jloc_oh, (((0,), (0,)), ((), ())),
                               preferred_element_type=f32)
        payload = jnp.concatenate(
            [xpAll, JAll, jnp.zeros(((N_DEV - 1) * CAP, PAD_L - D - E_LOC),
                                    f32)],
            axis=1).astype(bf16)
        disp_rdmas = []
        for k in far_first:
            pack_ref[k - 1] = payload[(k - 1) * CAP:k * CAP]
            rdma = pltpu.make_async_remote_copy(
                src_ref=pack_ref.at[k - 1],
                dst_ref=disp_ref.at[k - 1],
                send_sem=dsend.at[k - 1],
                recv_sem=drecv.at[k - 1],
                device_id=(lax.rem(my + k, N_DEV),),
                device_id_type=pl.DeviceIdType.MESH,
            )
            rdma.start()
            disp_rdmas.append(rdma)

        acc = jnp.dot(xv, sW_ref[:, :], preferred_element_type=f32)
        for j in range(E_LOC):
            gid = my * E_LOC + j
            cj = jnp.where(eidx == gid, gp, 0.0)
            acc = acc + jnp.dot(xv * cj, eW_ref[j],
                                preferred_element_type=f32)

        y_rdmas = []
        for k in near_first:
            src = lax.rem(my + (N_DEV - k), N_DEV)
            recv = pltpu.make_async_remote_copy(
                src_ref=pack_ref.at[k - 1],
                dst_ref=disp_ref.at[k - 1],
                send_sem=dsend.at[k - 1],
                recv_sem=drecv.at[k - 1],
                device_id=(src,),
                device_id_type=pl.DeviceIdType.MESH,
            )
            recv.wait_recv()
            pr = disp_ref[k - 1]
            xr = pr[:, :D].astype(f32)
            y = jnp.zeros((CAP, H), f32)
            for j in range(E_LOC):
                cj = pr[:, D + j:D + j + 1].astype(f32)
                y = y + jnp.dot(xr * cj, eW_ref[j],
                                preferred_element_type=f32)
            yout_ref[k - 1] = y.astype(bf16)
            rdma = pltpu.make_async_remote_copy(
                src_ref=yout_ref.at[k - 1],
                dst_ref=yin_ref.at[k - 1],
                send_sem=ysend.at[k - 1],
                recv_sem=yrecv.at[k - 1],
                device_id=(src,),
                device_id_type=pl.DeviceIdType.MESH,
            )
            rdma.start()
            y_rdmas.append(rdma)

        for k in near_first:
            recv = pltpu.make_async_remote_copy(
                src_ref=yout_ref.at[k - 1],
                dst_ref=yin_ref.at[k - 1],
                send_sem=ysend.at[k - 1],
                recv_sem=yrecv.at[k - 1],
                device_id=(lax.rem(my + k, N_DEV),),
                device_id_type=pl.DeviceIdType.MESH,
            )
            recv.wait_recv()
        ysAll = jnp.concatenate(
            [yin_ref[k - 1].astype(f32) for k in range(1, N_DEV)],
            axis=0)
        acc = acc + jnp.dot(PtAll, ysAll, preferred_element_type=f32)

        out_ref[:, :] = acc

        for r in disp_rdmas:
            r.wait_send()
        for r in y_rdmas:
            r.wait_send()

    return pl.pallas_call(
        body,
        out_shape=jax.ShapeDtypeStruct((T, H), jnp.float32),
        in_specs=[pl.BlockSpec(memory_space=pltpu.VMEM)] * 5,
        out_specs=pl.BlockSpec(memory_space=pltpu.VMEM),
        scratch_shapes=[
            pltpu.VMEM((N_DEV - 1, CAP, PAD_L), jnp.bfloat16),
            pltpu.VMEM((N_DEV - 1, CAP, PAD_L), jnp.bfloat16),
            pltpu.VMEM((N_DEV - 1, CAP, H), jnp.bfloat16),
            pltpu.VMEM((N_DEV - 1, CAP, H), jnp.bfloat16),
            pltpu.SemaphoreType.DMA((N_DEV - 1,)),
            pltpu.SemaphoreType.DMA((N_DEV - 1,)),
            pltpu.SemaphoreType.DMA((N_DEV - 1,)),
            pltpu.SemaphoreType.DMA((N_DEV - 1,)),
        ],
        compiler_params=pltpu.CompilerParams(
            collective_id=0,
            vmem_limit_bytes=96 * 1024 * 1024,
        ),
    )(x, router_W, route_idx, expert_W, shared_W)
